# baseline (speedup 1.0000x reference)
import functools

import jax
import jax.numpy as jnp
from jax import lax
from jax.experimental import pallas as pl
from jax.experimental.pallas import tpu as pltpu
from jax.experimental.pallas import tpu_sc as plsc

_L = 16


@functools.partial(jax.jit, static_argnums=(3, 4, 5))
def _project(img_1d, tvals, consts, Bn, n_row, n_col):
    npix = n_row * n_col
    n_ray, width = tvals.shape
    nseg = width - 1

    mesh = plsc.VectorSubcoreMesh(core_axis_name="c", subcore_axis_name="s")
    n_workers = mesh.num_cores * mesh.num_subcores
    chunks_per_batch = n_workers // Bn
    rays_per_worker = n_ray // chunks_per_batch
    rchunk = 32
    n_chunks = rays_per_worker // rchunk
    n_sub = rchunk // _L

    @functools.partial(
        pl.kernel,
        out_type=jax.ShapeDtypeStruct((Bn * n_ray,), jnp.float32),
        mesh=mesh,
        scratch_types=[
            pltpu.VMEM((npix,), jnp.float32),
            pltpu.VMEM((rchunk, width), jnp.float32),
            pltpu.VMEM((rchunk, width), jnp.float32),
            pltpu.VMEM((rchunk, 8), jnp.float32),
            pltpu.VMEM((rchunk, 8), jnp.float32),
            pltpu.VMEM((rays_per_worker,), jnp.float32),
            pltpu.SemaphoreType.DMA,
            pltpu.SemaphoreType.DMA,
            pltpu.SemaphoreType.DMA,
            pltpu.SemaphoreType.DMA,
        ],
        compiler_params=pltpu.CompilerParams(use_tc_tiling_on_sc=False,
                                             needs_layout_passes=False),
    )
    def proj(img_hbm, tv_hbm, cst_hbm, out_hbm,
             img_v, tv0, tv1, c0, c1, out_v, st0, sc0, st1, sc1):
        nc = mesh.num_cores
        wid = lax.axis_index("s") * nc + lax.axis_index("c")
        batch = wid // chunks_per_batch
        ray0 = (wid % chunks_per_batch) * rays_per_worker

        pltpu.sync_copy(img_hbm.at[pl.ds(batch * npix, npix)], img_v)

        lanes = lax.iota(jnp.int32, _L)
        zeros_i = jnp.zeros((_L,), jnp.int32)
        fzero = jnp.zeros((_L,), jnp.float32)

        def issue(ci, tvb, cvb, st, sc2):
            base = ray0 + ci * rchunk
            pltpu.make_async_copy(
                tv_hbm.at[pl.ds(base, rchunk)], tvb, st).start()
            pltpu.make_async_copy(
                cst_hbm.at[pl.ds(base, rchunk)], cvb, sc2).start()

        def wait(tvb, cvb, st, sc2):
            pltpu.make_async_copy(
                tv_hbm.at[pl.ds(0, rchunk)], tvb, st).wait()
            pltpu.make_async_copy(
                cst_hbm.at[pl.ds(0, rchunk)], cvb, sc2).wait()

        def process(ci, tv_v, c_v):
            for sub in range(n_sub):
                rid = lanes + (sub * _L)
                qsr = plsc.load_gather(c_v, [rid, zeros_i])
                qsc = plsc.load_gather(c_v, [rid, zeros_i + 1])
                dr = plsc.load_gather(c_v, [rid, zeros_i + 2])
                dc = plsc.load_gather(c_v, [rid, zeros_i + 3])
                rl = plsc.load_gather(c_v, [rid, zeros_i + 4])

                eps = 1e-9
                drs = jnp.where(jnp.abs(dr) < eps, eps, dr)
                dcs = jnp.where(jnp.abs(dc) < eps, eps, dc)
                tr0 = (0.0 - qsr) / drs
                tr1 = (float(n_row) - qsr) / drs
                tc0 = (0.0 - qsc) / dcs
                tc1 = (float(n_col) - qsc) / dcs
                t_en = jnp.maximum(jnp.maximum(jnp.minimum(tr0, tr1),
                                               jnp.minimum(tc0, tc1)),
                                   0.0) - 1e-4
                t_ex = jnp.minimum(jnp.minimum(jnp.maximum(tr0, tr1),
                                               jnp.maximum(tc0, tc1)),
                                   1.0) + 1e-4

                cnt_le = jnp.zeros((_L,), jnp.int32)
                cnt_lt = jnp.zeros((_L,), jnp.int32)
                step = 512
                while step >= 1:
                    cand = cnt_le + step
                    okc = cand <= width
                    probe = plsc.load_gather(
                        tv_v, [rid, jnp.minimum(cand, width) - 1])
                    cnt_le = jnp.where(okc & (probe <= t_en), cand, cnt_le)
                    cand2 = cnt_lt + step
                    okc2 = cand2 <= width
                    probe2 = plsc.load_gather(
                        tv_v, [rid, jnp.minimum(cand2, width) - 1])
                    cnt_lt = jnp.where(okc2 & (probe2 < t_ex), cand2, cnt_lt)
                    step //= 2
                blk_lo = jnp.min(jnp.maximum(cnt_le - 1, 0))
                blk_end = jnp.max(jnp.minimum(cnt_lt, nseg))

                t0 = plsc.load_gather(
                    tv_v, [rid, jnp.full((_L,), blk_lo, jnp.int32)])

                def seg_contrib(t_lo, t_hi, live, qsr=qsr, qsc=qsc,
                                dr=dr, dc=dc, rl=rl):
                    tm = 0.5 * (t_lo + t_hi)
                    pr = qsr + tm * dr
                    pc = qsc + tm * dc
                    seg = (t_hi - t_lo) * rl
                    inb = ((pr >= 0.0) & (pr < float(n_row))
                           & (pc >= 0.0) & (pc < float(n_col)))
                    if live is not None:
                        inb = inb & live
                    rs = jnp.clip(pr, 0.0, float(n_row - 1)).astype(jnp.int32)
                    cs = jnp.clip(pc, 0.0, float(n_col - 1)).astype(jnp.int32)
                    vals = plsc.load_gather(img_v, [rs * n_col + cs])
                    return vals * jnp.where(inb, seg, 0.0)

                def seg4_body(k, carry, rid=rid, tv_v=tv_v,
                              blk_lo=blk_lo, blk_end=blk_end):
                    t_c, acc = carry
                    s = blk_lo + 4 * k
                    t1 = plsc.load_gather(
                        tv_v, [rid, jnp.full((_L,), s + 1, jnp.int32)])
                    t2 = plsc.load_gather(
                        tv_v, [rid, jnp.full((_L,),
                                             jnp.minimum(s + 2, nseg),
                                             jnp.int32)])
                    t3 = plsc.load_gather(
                        tv_v, [rid, jnp.full((_L,),
                                             jnp.minimum(s + 3, nseg),
                                             jnp.int32)])
                    t4 = plsc.load_gather(
                        tv_v, [rid, jnp.full((_L,),
                                             jnp.minimum(s + 4, nseg),
                                             jnp.int32)])
                    acc = acc + seg_contrib(t_c, t1, None)
                    acc = acc + seg_contrib(t1, t2, s + 1 < blk_end)
                    acc = acc + seg_contrib(t2, t3, s + 2 < blk_end)
                    acc = acc + seg_contrib(t3, t4, s + 3 < blk_end)
                    return t4, acc

                n_q = (jnp.maximum(blk_end - blk_lo, 0) + 3) // 4
                _, acc = lax.fori_loop(0, n_q, seg4_body, (t0, fzero))
                out_v[pl.ds(ci * rchunk + sub * _L, _L)] = acc

        issue(0, tv0, c0, st0, sc0)

        def pair_body(gi, carry):
            ci0 = gi * 2
            wait(tv0, c0, st0, sc0)
            issue(ci0 + 1, tv1, c1, st1, sc1)
            process(ci0, tv0, c0)
            wait(tv1, c1, st1, sc1)

            @pl.when(ci0 + 2 < n_chunks)
            def _():
                issue(ci0 + 2, tv0, c0, st0, sc0)

            process(ci0 + 1, tv1, c1)
            return carry

        lax.fori_loop(0, n_chunks // 2, pair_body, 0)
        if n_chunks % 2:
            wait(tv0, c0, st0, sc0)
            process(n_chunks - 1, tv0, c0)
        pltpu.sync_copy(
            out_v, out_hbm.at[pl.ds(batch * n_ray + ray0, rays_per_worker)])

    return proj(img_1d, tvals, consts).reshape(Bn, n_ray)


def kernel(image, tvals, M, b, src, dst):
    squeeze = image.ndim == 2
    img = image[None] if squeeze else image
    Bn, n_row, n_col = img.shape

    Minv = jnp.linalg.inv(M)
    qs = (src - b[None, :]) @ Minv.T
    qd = (dst - b[None, :]) @ Minv.T
    d = qd - qs
    rl = jnp.linalg.norm(dst - src, axis=1)
    zero = jnp.zeros_like(rl)
    consts = jnp.stack([qs[:, 0], qs[:, 1], d[:, 0], d[:, 1], rl,
                        zero, zero, zero], axis=1)

    img_1d = img.reshape(Bn * n_row * n_col).astype(jnp.float32)
    sino = _project(img_1d, tvals.astype(jnp.float32), consts,
                    Bn, n_row, n_col)
    return sino[0] if squeeze else sino


# speedup vs baseline: 1.0490x; 1.0490x over previous
import functools

import jax
import jax.numpy as jnp
from jax import lax
from jax.experimental import pallas as pl
from jax.experimental.pallas import tpu as pltpu
from jax.experimental.pallas import tpu_sc as plsc

_L = 16


@functools.partial(jax.jit, static_argnums=(3, 4, 5))
def _project(img_1d, tvals, consts, Bn, n_row, n_col):
    npix = n_row * n_col
    n_ray, width = tvals.shape
    nseg = width - 1

    mesh = plsc.VectorSubcoreMesh(core_axis_name="c", subcore_axis_name="s")
    n_workers = mesh.num_cores * mesh.num_subcores
    chunks_per_batch = n_workers // Bn
    rays_per_worker = n_ray // chunks_per_batch
    rchunk = 32
    n_chunks = rays_per_worker // rchunk
    n_sub = rchunk // _L

    @functools.partial(
        pl.kernel,
        out_type=jax.ShapeDtypeStruct((Bn * n_ray,), jnp.float32),
        mesh=mesh,
        scratch_types=[
            pltpu.VMEM((npix,), jnp.float32),
            pltpu.VMEM((rchunk, width), jnp.float32),
            pltpu.VMEM((rchunk, width), jnp.float32),
            pltpu.VMEM((rchunk, 8), jnp.float32),
            pltpu.VMEM((rchunk, 8), jnp.float32),
            pltpu.VMEM((rays_per_worker,), jnp.float32),
            pltpu.SemaphoreType.DMA,
            pltpu.SemaphoreType.DMA,
            pltpu.SemaphoreType.DMA,
            pltpu.SemaphoreType.DMA,
        ],
        compiler_params=pltpu.CompilerParams(use_tc_tiling_on_sc=False,
                                             needs_layout_passes=False),
    )
    def proj(img_hbm, tv_hbm, cst_hbm, out_hbm,
             img_v, tv0, tv1, c0, c1, out_v, st0, sc0, st1, sc1):
        nc = mesh.num_cores
        wid = lax.axis_index("s") * nc + lax.axis_index("c")
        batch = wid // chunks_per_batch
        ray0 = (wid % chunks_per_batch) * rays_per_worker

        pltpu.sync_copy(img_hbm.at[pl.ds(batch * npix, npix)], img_v)

        lanes = lax.iota(jnp.int32, _L)
        zeros_i = jnp.zeros((_L,), jnp.int32)
        fzero = jnp.zeros((_L,), jnp.float32)

        def issue(ci, tvb, cvb, st, sc2):
            base = ray0 + ci * rchunk
            pltpu.make_async_copy(
                tv_hbm.at[pl.ds(base, rchunk)], tvb, st).start()
            pltpu.make_async_copy(
                cst_hbm.at[pl.ds(base, rchunk)], cvb, sc2).start()

        def wait(tvb, cvb, st, sc2):
            pltpu.make_async_copy(
                tv_hbm.at[pl.ds(0, rchunk)], tvb, st).wait()
            pltpu.make_async_copy(
                cst_hbm.at[pl.ds(0, rchunk)], cvb, sc2).wait()

        def process(ci, tv_v, c_v):
            for sub in range(n_sub):
                rid = lanes + (sub * _L)
                qsr = plsc.load_gather(c_v, [rid, zeros_i])
                qsc = plsc.load_gather(c_v, [rid, zeros_i + 1])
                dr = plsc.load_gather(c_v, [rid, zeros_i + 2])
                dc = plsc.load_gather(c_v, [rid, zeros_i + 3])
                rl = plsc.load_gather(c_v, [rid, zeros_i + 4])

                eps = 1e-9
                drs = jnp.where(jnp.abs(dr) < eps, eps, dr)
                dcs = jnp.where(jnp.abs(dc) < eps, eps, dc)
                tr0 = (0.0 - qsr) / drs
                tr1 = (float(n_row) - qsr) / drs
                tc0 = (0.0 - qsc) / dcs
                tc1 = (float(n_col) - qsc) / dcs
                t_en = jnp.maximum(jnp.maximum(jnp.minimum(tr0, tr1),
                                               jnp.minimum(tc0, tc1)),
                                   0.0) - 1e-4
                t_ex = jnp.minimum(jnp.minimum(jnp.maximum(tr0, tr1),
                                               jnp.maximum(tc0, tc1)),
                                   1.0) + 1e-4

                cnt_le = jnp.zeros((_L,), jnp.int32)
                cnt_lt = jnp.zeros((_L,), jnp.int32)
                step = 512
                while step >= 1:
                    cand = cnt_le + step
                    okc = cand <= width
                    probe = plsc.load_gather(
                        tv_v, [rid, jnp.minimum(cand, width) - 1])
                    cnt_le = jnp.where(okc & (probe <= t_en), cand, cnt_le)
                    cand2 = cnt_lt + step
                    okc2 = cand2 <= width
                    probe2 = plsc.load_gather(
                        tv_v, [rid, jnp.minimum(cand2, width) - 1])
                    cnt_lt = jnp.where(okc2 & (probe2 < t_ex), cand2, cnt_lt)
                    step //= 2
                blk_lo = jnp.min(jnp.maximum(cnt_le - 1, 0))
                blk_end = jnp.max(jnp.minimum(cnt_lt, nseg))

                t0 = plsc.load_gather(
                    tv_v, [rid, jnp.full((_L,), blk_lo, jnp.int32)])

                hdr = 0.5 * dr
                hdc = 0.5 * dc

                def seg_body(s, carry, rid=rid, tv_v=tv_v, qsr=qsr,
                             qsc=qsc, hdr=hdr, hdc=hdc, rl=rl):
                    t_cur, acc = carry
                    col = jnp.full((_L,), s + 1, jnp.int32)
                    t_nxt = plsc.load_gather(tv_v, [rid, col])
                    u = t_cur + t_nxt
                    pr = qsr + u * hdr
                    pc = qsc + u * hdc
                    seg = (t_nxt - t_cur) * rl
                    if n_row == n_col:
                        inb = ((jnp.minimum(pr, pc) >= 0.0)
                               & (jnp.maximum(pr, pc) < float(n_row)))
                    else:
                        inb = ((pr >= 0.0) & (pr < float(n_row))
                               & (pc >= 0.0) & (pc < float(n_col)))
                    rs = jnp.clip(pr, 0.0, float(n_row - 1)).astype(jnp.int32)
                    cs = jnp.clip(pc, 0.0, float(n_col - 1)).astype(jnp.int32)
                    flat = rs * n_col + cs
                    vals = plsc.load_gather(img_v, [flat])
                    w = jnp.where(inb, seg, 0.0)
                    return t_nxt, acc + vals * w

                _, acc = lax.fori_loop(blk_lo, blk_end, seg_body,
                                       (t0, fzero))
                out_v[pl.ds(ci * rchunk + sub * _L, _L)] = acc

        issue(0, tv0, c0, st0, sc0)

        def pair_body(gi, carry):
            ci0 = gi * 2
            wait(tv0, c0, st0, sc0)
            issue(ci0 + 1, tv1, c1, st1, sc1)
            process(ci0, tv0, c0)
            wait(tv1, c1, st1, sc1)

            @pl.when(ci0 + 2 < n_chunks)
            def _():
                issue(ci0 + 2, tv0, c0, st0, sc0)

            process(ci0 + 1, tv1, c1)
            return carry

        lax.fori_loop(0, n_chunks // 2, pair_body, 0)
        if n_chunks % 2:
            wait(tv0, c0, st0, sc0)
            process(n_chunks - 1, tv0, c0)
        pltpu.sync_copy(
            out_v, out_hbm.at[pl.ds(batch * n_ray + ray0, rays_per_worker)])

    return proj(img_1d, tvals, consts).reshape(Bn, n_ray)


def kernel(image, tvals, M, b, src, dst):
    squeeze = image.ndim == 2
    img = image[None] if squeeze else image
    Bn, n_row, n_col = img.shape

    Minv = jnp.linalg.inv(M)
    qs = (src - b[None, :]) @ Minv.T
    qd = (dst - b[None, :]) @ Minv.T
    d = qd - qs
    rl = jnp.linalg.norm(dst - src, axis=1)
    zero = jnp.zeros_like(rl)
    consts = jnp.stack([qs[:, 0], qs[:, 1], d[:, 0], d[:, 1], rl,
                        zero, zero, zero], axis=1)

    img_1d = img.reshape(Bn * n_row * n_col).astype(jnp.float32)
    sino = _project(img_1d, tvals.astype(jnp.float32), consts,
                    Bn, n_row, n_col)
    return sino[0] if squeeze else sino


# speedup vs baseline: 1.1010x; 1.0496x over previous
import functools

import jax
import jax.numpy as jnp
from jax import lax
from jax.experimental import pallas as pl
from jax.experimental.pallas import tpu as pltpu
from jax.experimental.pallas import tpu_sc as plsc

_L = 16


@functools.partial(jax.jit, static_argnums=(3, 4, 5))
def _project(img_1d, tvals, consts, Bn, n_row, n_col):
    npix = n_row * n_col
    n_ray, width = tvals.shape
    nseg = width - 1

    mesh = plsc.VectorSubcoreMesh(core_axis_name="c", subcore_axis_name="s")
    n_workers = mesh.num_cores * mesh.num_subcores
    chunks_per_batch = n_workers // Bn
    rays_per_worker = n_ray // chunks_per_batch
    rchunk = 32
    n_chunks = rays_per_worker // rchunk
    n_sub = rchunk // _L

    @functools.partial(
        pl.kernel,
        out_type=jax.ShapeDtypeStruct((Bn * n_ray,), jnp.float32),
        mesh=mesh,
        scratch_types=[
            pltpu.VMEM((npix,), jnp.float32),
            pltpu.VMEM((rchunk, width), jnp.float32),
            pltpu.VMEM((rchunk, width), jnp.float32),
            pltpu.VMEM((rchunk, 8), jnp.float32),
            pltpu.VMEM((rchunk, 8), jnp.float32),
            pltpu.VMEM((rays_per_worker,), jnp.float32),
            pltpu.SemaphoreType.DMA,
            pltpu.SemaphoreType.DMA,
            pltpu.SemaphoreType.DMA,
            pltpu.SemaphoreType.DMA,
        ],
        compiler_params=pltpu.CompilerParams(use_tc_tiling_on_sc=False,
                                             needs_layout_passes=False),
    )
    def proj(img_hbm, tv_hbm, cst_hbm, out_hbm,
             img_v, tv0, tv1, c0, c1, out_v, st0, sc0, st1, sc1):
        nc = mesh.num_cores
        wid = lax.axis_index("s") * nc + lax.axis_index("c")
        batch = wid // chunks_per_batch
        ray0 = (wid % chunks_per_batch) * rays_per_worker

        pltpu.sync_copy(img_hbm.at[pl.ds(batch * npix, npix)], img_v)

        lanes = lax.iota(jnp.int32, _L)
        zeros_i = jnp.zeros((_L,), jnp.int32)
        fzero = jnp.zeros((_L,), jnp.float32)

        def issue(ci, tvb, cvb, st, sc2):
            base = ray0 + ci * rchunk
            pltpu.make_async_copy(
                tv_hbm.at[pl.ds(base, rchunk)], tvb, st).start()
            pltpu.make_async_copy(
                cst_hbm.at[pl.ds(base, rchunk)], cvb, sc2).start()

        def wait(tvb, cvb, st, sc2):
            pltpu.make_async_copy(
                tv_hbm.at[pl.ds(0, rchunk)], tvb, st).wait()
            pltpu.make_async_copy(
                cst_hbm.at[pl.ds(0, rchunk)], cvb, sc2).wait()

        def process(ci, tv_v, c_v):
            for sub in range(n_sub):
                rid = lanes + (sub * _L)
                qsr = plsc.load_gather(c_v, [rid, zeros_i])
                qsc = plsc.load_gather(c_v, [rid, zeros_i + 1])
                dr = plsc.load_gather(c_v, [rid, zeros_i + 2])
                dc = plsc.load_gather(c_v, [rid, zeros_i + 3])
                rl = plsc.load_gather(c_v, [rid, zeros_i + 4])

                eps = 1e-9
                drs = jnp.where(jnp.abs(dr) < eps, eps, dr)
                dcs = jnp.where(jnp.abs(dc) < eps, eps, dc)
                tr0 = (0.0 - qsr) / drs
                tr1 = (float(n_row) - qsr) / drs
                tc0 = (0.0 - qsc) / dcs
                tc1 = (float(n_col) - qsc) / dcs
                t_en = jnp.maximum(jnp.maximum(jnp.minimum(tr0, tr1),
                                               jnp.minimum(tc0, tc1)),
                                   0.0) - 1e-4
                t_ex = jnp.minimum(jnp.minimum(jnp.maximum(tr0, tr1),
                                               jnp.maximum(tc0, tc1)),
                                   1.0) + 1e-4

                t_en_in = t_en + 2e-4
                t_ex_in = t_ex - 2e-4

                cnt_le = jnp.zeros((_L,), jnp.int32)
                cnt_lt = jnp.zeros((_L,), jnp.int32)
                cnt_lt_in = jnp.zeros((_L,), jnp.int32)
                cnt_le_in = jnp.zeros((_L,), jnp.int32)
                step = 512
                while step >= 1:
                    for which in range(4):
                        cnt = (cnt_le, cnt_lt, cnt_lt_in, cnt_le_in)[which]
                        cand = cnt + step
                        okc = cand <= width
                        probe = plsc.load_gather(
                            tv_v, [rid, jnp.minimum(cand, width) - 1])
                        if which == 0:
                            take = okc & (probe <= t_en)
                            cnt_le = jnp.where(take, cand, cnt_le)
                        elif which == 1:
                            take = okc & (probe < t_ex)
                            cnt_lt = jnp.where(take, cand, cnt_lt)
                        elif which == 2:
                            take = okc & (probe < t_en_in)
                            cnt_lt_in = jnp.where(take, cand, cnt_lt_in)
                        else:
                            take = okc & (probe <= t_ex_in)
                            cnt_le_in = jnp.where(take, cand, cnt_le_in)
                    step //= 2
                blk_lo = jnp.min(jnp.maximum(cnt_le - 1, 0))
                blk_end = jnp.max(jnp.minimum(cnt_lt, nseg))
                in_lo = jnp.max(cnt_lt_in)
                in_end = jnp.min(cnt_le_in) - 1
                in_lo = jnp.clip(in_lo, blk_lo, blk_end)
                in_end = jnp.clip(in_end, in_lo, blk_end)

                def seg_body(s, carry, rid=rid, tv_v=tv_v, qsr=qsr,
                             qsc=qsc, dr=dr, dc=dc, rl=rl):
                    t_cur, acc = carry
                    col = jnp.full((_L,), s + 1, jnp.int32)
                    t_nxt = plsc.load_gather(tv_v, [rid, col])
                    tm = 0.5 * (t_cur + t_nxt)
                    pr = qsr + tm * dr
                    pc = qsc + tm * dc
                    seg = (t_nxt - t_cur) * rl
                    inb = ((pr >= 0.0) & (pr < float(n_row))
                           & (pc >= 0.0) & (pc < float(n_col)))
                    rs = jnp.clip(pr, 0.0, float(n_row - 1)).astype(jnp.int32)
                    cs = jnp.clip(pc, 0.0, float(n_col - 1)).astype(jnp.int32)
                    flat = rs * n_col + cs
                    vals = plsc.load_gather(img_v, [flat])
                    w = jnp.where(inb, seg, 0.0)
                    return t_nxt, acc + vals * w

                def fast_body(s, carry, rid=rid, tv_v=tv_v, qsr=qsr,
                              qsc=qsc, dr=dr, dc=dc, rl=rl):
                    t_cur, acc = carry
                    col = jnp.full((_L,), s + 1, jnp.int32)
                    t_nxt = plsc.load_gather(tv_v, [rid, col])
                    tm = 0.5 * (t_cur + t_nxt)
                    pr = qsr + tm * dr
                    pc = qsc + tm * dc
                    seg = (t_nxt - t_cur) * rl
                    flat = pr.astype(jnp.int32) * n_col + pc.astype(jnp.int32)
                    vals = plsc.load_gather(img_v, [flat])
                    return t_nxt, acc + vals * seg

                t0 = plsc.load_gather(
                    tv_v, [rid, jnp.full((_L,), blk_lo, jnp.int32)])
                t_c, acc = lax.fori_loop(blk_lo, in_lo, seg_body,
                                         (t0, fzero))
                t_c, acc = lax.fori_loop(in_lo, in_end, fast_body,
                                         (t_c, acc))
                _, acc = lax.fori_loop(in_end, blk_end, seg_body,
                                       (t_c, acc))
                out_v[pl.ds(ci * rchunk + sub * _L, _L)] = acc

        issue(0, tv0, c0, st0, sc0)

        def pair_body(gi, carry):
            ci0 = gi * 2
            wait(tv0, c0, st0, sc0)
            issue(ci0 + 1, tv1, c1, st1, sc1)
            process(ci0, tv0, c0)
            wait(tv1, c1, st1, sc1)

            @pl.when(ci0 + 2 < n_chunks)
            def _():
                issue(ci0 + 2, tv0, c0, st0, sc0)

            process(ci0 + 1, tv1, c1)
            return carry

        lax.fori_loop(0, n_chunks // 2, pair_body, 0)
        if n_chunks % 2:
            wait(tv0, c0, st0, sc0)
            process(n_chunks - 1, tv0, c0)
        pltpu.sync_copy(
            out_v, out_hbm.at[pl.ds(batch * n_ray + ray0, rays_per_worker)])

    return proj(img_1d, tvals, consts).reshape(Bn, n_ray)


def kernel(image, tvals, M, b, src, dst):
    squeeze = image.ndim == 2
    img = image[None] if squeeze else image
    Bn, n_row, n_col = img.shape

    Minv = jnp.linalg.inv(M)
    qs = (src - b[None, :]) @ Minv.T
    qd = (dst - b[None, :]) @ Minv.T
    d = qd - qs
    rl = jnp.linalg.norm(dst - src, axis=1)
    zero = jnp.zeros_like(rl)
    consts = jnp.stack([qs[:, 0], qs[:, 1], d[:, 0], d[:, 1], rl,
                        zero, zero, zero], axis=1)

    img_1d = img.reshape(Bn * n_row * n_col).astype(jnp.float32)
    sino = _project(img_1d, tvals.astype(jnp.float32), consts,
                    Bn, n_row, n_col)
    return sino[0] if squeeze else sino


# speedup vs baseline: 1.1164x; 1.0140x over previous
import functools

import jax
import jax.numpy as jnp
from jax import lax
from jax.experimental import pallas as pl
from jax.experimental.pallas import tpu as pltpu
from jax.experimental.pallas import tpu_sc as plsc

_L = 16


@functools.partial(jax.jit, static_argnums=(3, 4, 5))
def _project(img_1d, tvals, consts, Bn, n_row, n_col):
    npix = n_row * n_col
    n_ray, width = tvals.shape
    nseg = width - 1

    mesh = plsc.VectorSubcoreMesh(core_axis_name="c", subcore_axis_name="s")
    n_workers = mesh.num_cores * mesh.num_subcores
    chunks_per_batch = n_workers // Bn
    rays_per_worker = n_ray // chunks_per_batch
    rchunk = 32
    n_chunks = rays_per_worker // rchunk
    n_sub = rchunk // _L

    @functools.partial(
        pl.kernel,
        out_type=jax.ShapeDtypeStruct((Bn * n_ray,), jnp.float32),
        mesh=mesh,
        scratch_types=[
            pltpu.VMEM((npix,), jnp.float32),
            pltpu.VMEM((rchunk, width), jnp.float32),
            pltpu.VMEM((rchunk, width), jnp.float32),
            pltpu.VMEM((rchunk, 8), jnp.float32),
            pltpu.VMEM((rchunk, 8), jnp.float32),
            pltpu.VMEM((rays_per_worker,), jnp.float32),
            pltpu.SemaphoreType.DMA,
            pltpu.SemaphoreType.DMA,
            pltpu.SemaphoreType.DMA,
            pltpu.SemaphoreType.DMA,
        ],
        compiler_params=pltpu.CompilerParams(use_tc_tiling_on_sc=False,
                                             needs_layout_passes=False),
    )
    def proj(img_hbm, tv_hbm, cst_hbm, out_hbm,
             img_v, tv0, tv1, c0, c1, out_v, st0, sc0, st1, sc1):
        nc = mesh.num_cores
        wid = lax.axis_index("s") * nc + lax.axis_index("c")
        batch = wid // chunks_per_batch
        ray0 = (wid % chunks_per_batch) * rays_per_worker

        pltpu.sync_copy(img_hbm.at[pl.ds(batch * npix, npix)], img_v)

        lanes = lax.iota(jnp.int32, _L)
        zeros_i = jnp.zeros((_L,), jnp.int32)
        fzero = jnp.zeros((_L,), jnp.float32)

        def issue(ci, tvb, cvb, st, sc2):
            base = ray0 + ci * rchunk
            pltpu.make_async_copy(
                tv_hbm.at[pl.ds(base, rchunk)], tvb, st).start()
            pltpu.make_async_copy(
                cst_hbm.at[pl.ds(base, rchunk)], cvb, sc2).start()

        def wait(tvb, cvb, st, sc2):
            pltpu.make_async_copy(
                tv_hbm.at[pl.ds(0, rchunk)], tvb, st).wait()
            pltpu.make_async_copy(
                cst_hbm.at[pl.ds(0, rchunk)], cvb, sc2).wait()

        def process(ci, tv_v, c_v):
            rids = [lanes + (sub * _L) for sub in range(n_sub)]
            qsr_l, qsc_l, dr_l, dc_l, rl_l = [], [], [], [], []
            blk_lo = blk_end = in_lo = in_end = None
            for sub in range(n_sub):
                rid = rids[sub]
                qsr = plsc.load_gather(c_v, [rid, zeros_i])
                qsc = plsc.load_gather(c_v, [rid, zeros_i + 1])
                dr = plsc.load_gather(c_v, [rid, zeros_i + 2])
                dc = plsc.load_gather(c_v, [rid, zeros_i + 3])
                rl = plsc.load_gather(c_v, [rid, zeros_i + 4])
                qsr_l.append(qsr)
                qsc_l.append(qsc)
                dr_l.append(dr)
                dc_l.append(dc)
                rl_l.append(rl)

                eps = 1e-9
                drs = jnp.where(jnp.abs(dr) < eps, eps, dr)
                dcs = jnp.where(jnp.abs(dc) < eps, eps, dc)
                tr0 = (0.0 - qsr) / drs
                tr1 = (float(n_row) - qsr) / drs
                tc0 = (0.0 - qsc) / dcs
                tc1 = (float(n_col) - qsc) / dcs
                t_en = jnp.maximum(jnp.maximum(jnp.minimum(tr0, tr1),
                                               jnp.minimum(tc0, tc1)),
                                   0.0) - 1e-4
                t_ex = jnp.minimum(jnp.minimum(jnp.maximum(tr0, tr1),
                                               jnp.maximum(tc0, tc1)),
                                   1.0) + 1e-4

                t_en_in = t_en + 2e-4
                t_ex_in = t_ex - 2e-4

                cnt_le = jnp.zeros((_L,), jnp.int32)
                cnt_lt = jnp.zeros((_L,), jnp.int32)
                cnt_lt_in = jnp.zeros((_L,), jnp.int32)
                cnt_le_in = jnp.zeros((_L,), jnp.int32)
                step = 512
                while step >= 1:
                    for which in range(4):
                        cnt = (cnt_le, cnt_lt, cnt_lt_in, cnt_le_in)[which]
                        cand = cnt + step
                        okc = cand <= width
                        probe = plsc.load_gather(
                            tv_v, [rid, jnp.minimum(cand, width) - 1])
                        if which == 0:
                            take = okc & (probe <= t_en)
                            cnt_le = jnp.where(take, cand, cnt_le)
                        elif which == 1:
                            take = okc & (probe < t_ex)
                            cnt_lt = jnp.where(take, cand, cnt_lt)
                        elif which == 2:
                            take = okc & (probe < t_en_in)
                            cnt_lt_in = jnp.where(take, cand, cnt_lt_in)
                        else:
                            take = okc & (probe <= t_ex_in)
                            cnt_le_in = jnp.where(take, cand, cnt_le_in)
                    step //= 2
                b_lo = jnp.min(jnp.maximum(cnt_le - 1, 0))
                b_end = jnp.max(jnp.minimum(cnt_lt, nseg))
                i_lo = jnp.max(cnt_lt_in)
                i_end = jnp.min(cnt_le_in) - 1
                if sub == 0:
                    blk_lo, blk_end, in_lo, in_end = b_lo, b_end, i_lo, i_end
                else:
                    blk_lo = jnp.minimum(blk_lo, b_lo)
                    blk_end = jnp.maximum(blk_end, b_end)
                    in_lo = jnp.maximum(in_lo, i_lo)
                    in_end = jnp.minimum(in_end, i_end)
            blk_end = jnp.maximum(blk_end, blk_lo)
            in_lo = jnp.clip(in_lo, blk_lo, blk_end)
            in_end = jnp.clip(in_end, in_lo, blk_end)

            def seg_step(t_cur, acc, col, sub):
                t_nxt = plsc.load_gather(tv_v, [rids[sub], col])
                tm = 0.5 * (t_cur + t_nxt)
                pr = qsr_l[sub] + tm * dr_l[sub]
                pc = qsc_l[sub] + tm * dc_l[sub]
                seg = (t_nxt - t_cur) * rl_l[sub]
                inb = ((pr >= 0.0) & (pr < float(n_row))
                       & (pc >= 0.0) & (pc < float(n_col)))
                rs = jnp.clip(pr, 0.0, float(n_row - 1)).astype(jnp.int32)
                cs = jnp.clip(pc, 0.0, float(n_col - 1)).astype(jnp.int32)
                vals = plsc.load_gather(img_v, [rs * n_col + cs])
                return t_nxt, acc + vals * jnp.where(inb, seg, 0.0)

            def fast_step(t_cur, acc, col, sub):
                t_nxt = plsc.load_gather(tv_v, [rids[sub], col])
                tm = 0.5 * (t_cur + t_nxt)
                pr = qsr_l[sub] + tm * dr_l[sub]
                pc = qsc_l[sub] + tm * dc_l[sub]
                seg = (t_nxt - t_cur) * rl_l[sub]
                flat = pr.astype(jnp.int32) * n_col + pc.astype(jnp.int32)
                vals = plsc.load_gather(img_v, [flat])
                return t_nxt, acc + vals * seg

            def multi(step_fn):
                def body(s, carry):
                    ts, accs = carry
                    col = jnp.full((_L,), s + 1, jnp.int32)
                    new = [step_fn(ts[h], accs[h], col, h)
                           for h in range(n_sub)]
                    return tuple(t for t, _ in new), tuple(a for _, a in new)
                return body

            col0 = jnp.full((_L,), blk_lo, jnp.int32)
            ts0 = tuple(plsc.load_gather(tv_v, [rids[h], col0])
                        for h in range(n_sub))
            accs0 = tuple(fzero for _ in range(n_sub))
            carry = lax.fori_loop(blk_lo, in_lo, multi(seg_step),
                                  (ts0, accs0))
            carry = lax.fori_loop(in_lo, in_end, multi(fast_step), carry)
            _, accs = lax.fori_loop(in_end, blk_end, multi(seg_step), carry)
            for sub in range(n_sub):
                out_v[pl.ds(ci * rchunk + sub * _L, _L)] = accs[sub]

        issue(0, tv0, c0, st0, sc0)

        def pair_body(gi, carry):
            ci0 = gi * 2
            wait(tv0, c0, st0, sc0)
            issue(ci0 + 1, tv1, c1, st1, sc1)
            process(ci0, tv0, c0)
            wait(tv1, c1, st1, sc1)

            @pl.when(ci0 + 2 < n_chunks)
            def _():
                issue(ci0 + 2, tv0, c0, st0, sc0)

            process(ci0 + 1, tv1, c1)
            return carry

        lax.fori_loop(0, n_chunks // 2, pair_body, 0)
        if n_chunks % 2:
            wait(tv0, c0, st0, sc0)
            process(n_chunks - 1, tv0, c0)
        pltpu.sync_copy(
            out_v, out_hbm.at[pl.ds(batch * n_ray + ray0, rays_per_worker)])

    return proj(img_1d, tvals, consts).reshape(Bn, n_ray)


def kernel(image, tvals, M, b, src, dst):
    squeeze = image.ndim == 2
    img = image[None] if squeeze else image
    Bn, n_row, n_col = img.shape

    Minv = jnp.linalg.inv(M)
    qs = (src - b[None, :]) @ Minv.T
    qd = (dst - b[None, :]) @ Minv.T
    d = qd - qs
    rl = jnp.linalg.norm(dst - src, axis=1)
    zero = jnp.zeros_like(rl)
    consts = jnp.stack([qs[:, 0], qs[:, 1], d[:, 0], d[:, 1], rl,
                        zero, zero, zero], axis=1)

    img_1d = img.reshape(Bn * n_row * n_col).astype(jnp.float32)
    sino = _project(img_1d, tvals.astype(jnp.float32), consts,
                    Bn, n_row, n_col)
    return sino[0] if squeeze else sino
